# baseline (speedup 1.0000x reference)
import jax
import jax.numpy as jnp
from jax import lax
from jax.experimental import pallas as pl
from jax.experimental.pallas import tpu as pltpu
from jax.experimental.pallas import tpu_sc as plsc

ROWS = 128
COLS = 32768
FILLER = 32

NC = 2
NS = 16
LANES = 16
RB = 8
N_BLOCKS = ROWS // RB

SCAN_CH = 256
N_SCAN_CH = COLS // SCAN_CH
THRESH = 0.5

MEMSET_CH = 4096


def _memset_body(o_ref):
    o_ref[...] = jnp.zeros_like(o_ref)


def _tc_zeros():
    return pl.pallas_call(
        _memset_body,
        out_shape=jax.ShapeDtypeStruct((ROWS, COLS), jnp.float32),
        grid=(COLS // MEMSET_CH,),
        out_specs=pl.BlockSpec((ROWS, MEMSET_CH), lambda i: (0, i)),
    )()


def _sc_update(refs):
    in_hbm, fp_hbm, out_hbm = refs
    mesh = plsc.VectorSubcoreMesh(
        core_axis_name="c", subcore_axis_name="s",
        num_cores=1, num_subcores=NS)

    @pl.core_map(
        mesh,
        compiler_params=pltpu.CompilerParams(needs_layout_passes=False))
    def _():
        c = lax.axis_index("c")
        s = lax.axis_index("s")
        row0 = s * RB

        def scoped(in_buf, fp_buf, out_buf, sem_in, sem_out):
            pltpu.async_copy(
                in_hbm.at[pl.ds(row0, RB), pl.ds(0, SCAN_CH)], in_buf,
                sem_in)
            pltpu.async_copy(
                fp_hbm.at[pl.ds(row0, RB), pl.ds(0, SCAN_CH)], fp_buf,
                sem_in)

            def scan_cond(state):
                ch = state[0]
                cnts = state[1:]
                cnt_min = cnts[0]
                for v in cnts[1:]:
                    cnt_min = jnp.minimum(cnt_min, v)
                return jnp.logical_and(cnt_min < FILLER, ch < N_SCAN_CH)

            def scan_body(state):
                ch = state[0]
                cnts = list(state[1:])
                start = pl.multiple_of(ch * SCAN_CH, SCAN_CH)

                @pl.when(ch > 0)
                def _():
                    pltpu.async_copy(
                        in_hbm.at[pl.ds(row0, RB), pl.ds(start, SCAN_CH)],
                        in_buf, sem_in)
                    pltpu.async_copy(
                        fp_hbm.at[pl.ds(row0, RB), pl.ds(start, SCAN_CH)],
                        fp_buf, sem_in)
                pltpu.make_async_copy(
                    in_hbm.at[pl.ds(row0, RB), pl.ds(start, SCAN_CH)],
                    in_buf, sem_in).wait()
                pltpu.make_async_copy(
                    fp_hbm.at[pl.ds(row0, RB), pl.ds(start, SCAN_CH)],
                    fp_buf, sem_in).wait()
                for rr in range(RB):
                    def vec_body(i, cnt, rr=rr):
                        vi = in_buf[rr, pl.ds(i * LANES, LANES)]
                        vf = fp_buf[rr, pl.ds(i * LANES, LANES)]
                        m = jnp.logical_and(vi > 0.0, vf > THRESH)
                        ones = jnp.where(m, jnp.float32(1.0),
                                         jnp.float32(0.0))
                        cs = plsc.cumsum(ones)
                        keep = jnp.logical_and(
                            m, (cnt.astype(jnp.float32) + cs)
                            <= jnp.float32(FILLER))
                        out_buf[rr, pl.ds(i * LANES, LANES)] = jnp.where(
                            keep, jnp.float32(1.0), jnp.float32(0.0))
                        return cnt + jnp.sum(ones).astype(jnp.int32)
                    cnts[rr] = lax.fori_loop(
                        0, SCAN_CH // LANES, vec_body, cnts[rr])
                pltpu.async_copy(
                    out_buf,
                    out_hbm.at[pl.ds(row0, RB), pl.ds(start, SCAN_CH)],
                    sem_out).wait()
                return (ch + 1, *cnts)

            lax.while_loop(scan_cond, scan_body, (0,) + (0,) * RB)

        @pl.when(c == 0)
        def _owner():
            pl.run_scoped(
                scoped,
                pltpu.VMEM((RB, SCAN_CH), jnp.float32),
                pltpu.VMEM((RB, SCAN_CH), jnp.float32),
                pltpu.VMEM((RB, SCAN_CH), jnp.float32),
                pltpu.SemaphoreType.DMA,
                pltpu.SemaphoreType.DMA,
            )


@jax.jit
def _fake_profile(inp, fp):
    zeros = _tc_zeros()
    _, _, out = pl.run_state(_sc_update)((inp, fp, zeros))
    return out


def kernel(input, fake_param):
    return _fake_profile(input, fake_param)
